# Initial kernel scaffold; baseline (speedup 1.0000x reference)
import jax
import jax.numpy as jnp
from jax.experimental import pallas as pl


def kernel(x, edge_index, graph_ids, W, attn_l, attn_r, W1, b1, W2, b2):
    raise NotImplementedError("write your pallas kernel here")


# speedup vs baseline: 34.2359x; 34.2359x over previous
import functools

import jax
import jax.numpy as jnp
from jax import lax
from jax.experimental import pallas as pl
from jax.experimental.pallas import tpu as pltpu
from jax.experimental.pallas import tpu_sc as plsc

_N = 10000
_E = 320000
_D = 128
_H = 8
_F = 32
_G = 32
_PH = 512
_SLOPE = 0.2

_HF = _H * _F
_HALF = _HF // 2
_ROW = 144
_ERW = 16
_NC, _NS = 2, 16
_K = 80
_EPT = _E // _NS
_NWIN = _EPT // _K
_ACCN = 10240


_BN = 1000


def _proj_body(x_ref, w_ref, al_ref, ar_ref, tlo_ref, thi_ref, er_ref):
    h = jnp.dot(x_ref[...], w_ref[...], preferred_element_type=jnp.float32)
    el = jnp.dot(h, al_ref[...], preferred_element_type=jnp.float32,
                 precision=lax.Precision.HIGHEST)
    er = jnp.dot(h, ar_ref[...], preferred_element_type=jnp.float32,
                 precision=lax.Precision.HIGHEST)
    pad12 = jnp.zeros((_BN, 12), jnp.float32)
    tlo_ref[...] = jnp.concatenate([h[:, :_HALF], el[:, :4], pad12], axis=1)
    thi_ref[...] = jnp.concatenate([h[:, _HALF:], el[:, 4:], pad12], axis=1)
    er_ref[...] = jnp.concatenate([er, jnp.zeros((_BN, 8), jnp.float32)],
                                  axis=1)


_proj = pl.pallas_call(
    _proj_body,
    grid=(_N // _BN,),
    in_specs=[
        pl.BlockSpec((_BN, _D), lambda i: (i, 0)),
        pl.BlockSpec((_D, _HF), lambda i: (0, 0)),
        pl.BlockSpec((_HF, _H), lambda i: (0, 0)),
        pl.BlockSpec((_HF, _H), lambda i: (0, 0)),
    ],
    out_specs=[
        pl.BlockSpec((_BN, _ROW), lambda i: (i, 0)),
        pl.BlockSpec((_BN, _ROW), lambda i: (i, 0)),
        pl.BlockSpec((_BN, _ERW), lambda i: (i, 0)),
    ],
    out_shape=[
        jax.ShapeDtypeStruct((_N, _ROW), jnp.float32),
        jax.ShapeDtypeStruct((_N, _ROW), jnp.float32),
        jax.ShapeDtypeStruct((_N, _ERW), jnp.float32),
    ],
)


def _exp16(e):
    t = e * 1.4426950408889634
    half = jnp.where(t >= 0.0, 0.5, -0.5)
    k = (t + half).astype(jnp.int32)
    kc = jnp.clip(k, -126, 127)
    z = (t - kc.astype(jnp.float32)) * 0.6931471805599453
    p = jnp.float32(1.0 / 720.0)
    p = p * z + jnp.float32(1.0 / 120.0)
    p = p * z + jnp.float32(1.0 / 24.0)
    p = p * z + jnp.float32(1.0 / 6.0)
    p = p * z + jnp.float32(0.5)
    p = p * z + jnp.float32(1.0)
    p = p * z + jnp.float32(1.0)
    scale = lax.bitcast_convert_type((kc + 127) << 23, jnp.float32)
    return p * scale


def _edge_body(tlo, thi, ertab, src_hbm, dst_hbm, out_hbm,
               src_v, dst_v, rows_v, er_v, msg_v, eex_v, acc,
               sem_r, sem_e):
    c = lax.axis_index("c")
    s = lax.axis_index("s")
    iota16 = lax.iota(jnp.int32, 16)
    zero16 = jnp.zeros((16,), jnp.float32)

    def _zrow(i, carry):
        for q in range(_ROW // 16):
            msg_v[i, pl.ds(q * 16, 16)] = zero16
        return carry

    lax.fori_loop(0, _K, _zrow, 0)

    def _zacc(k, carry):
        pltpu.sync_copy(msg_v, acc.at[pl.ds(s * 640 + k * _K, _K)])
        return carry

    lax.fori_loop(0, 640 // _K, _zacc, 0)
    plsc.subcore_barrier()

    hoff = c * 4

    def _window(w, carry):
        base = s * _EPT + w * _K
        pltpu.sync_copy(src_hbm.at[pl.ds(base, _K)], src_v)
        pltpu.sync_copy(dst_hbm.at[pl.ds(base, _K)], dst_v)
        cp_er = pltpu.async_copy(ertab.at[dst_v], er_v, sem_e)

        @pl.when(c == 0)
        def _():
            pltpu.async_copy(tlo.at[src_v], rows_v, sem_r).wait()

        @pl.when(c == 1)
        def _():
            pltpu.async_copy(thi.at[src_v], rows_v, sem_r).wait()

        cp_er.wait()

        def _sub(j, carry2):
            rbase = j * 16
            ridx = rbase + iota16
            exs = []
            for hh in range(4):
                col = jnp.full((16,), _HALF + hh, jnp.int32)
                elh = plsc.load_gather(rows_v, [ridx, col])
                ecol = jnp.full((16,), hh, jnp.int32) + hoff
                erh = plsc.load_gather(er_v, [ridx, ecol])
                e = elh + erh
                e = jnp.where(e > 0.0, e, _SLOPE * e)
                ex = _exp16(e)
                eex_v[hh] = ex
                exs.append(ex)
            for l in range(16):
                r = rbase + l
                lv = jnp.full((16,), l, jnp.int32)
                sp = [jnp.full((16,),
                               jnp.sum(jnp.where(iota16 == l, exs[hh], 0.0)),
                               jnp.float32)
                      for hh in range(4)]
                for q in range(8):
                    hv = rows_v[r, pl.ds(q * 16, 16)]
                    msg_v[r, pl.ds(q * 16, 16)] = hv * sp[q // 2]
                v9 = plsc.load_gather(eex_v, [iota16 & 3, lv])
                v9 = jnp.where(iota16 < 4, v9, 0.0)
                msg_v[r, pl.ds(_HALF, 16)] = v9
            return carry2

        lax.fori_loop(0, _K // 16, _sub, 0)
        pltpu.sync_copy(msg_v, acc.at[dst_v], add=True)
        return carry

    lax.fori_loop(0, _NWIN, _window, 0)
    plsc.subcore_barrier()

    @pl.when(s == 0)
    def _():
        pltpu.sync_copy(acc.at[pl.ds(0, _N)], out_hbm.at[c])


@functools.cache
def _make_edge():
  return functools.partial(
    pl.kernel,
    out_type=jax.ShapeDtypeStruct((_NC, _N, _ROW), jnp.float32),
    mesh=plsc.VectorSubcoreMesh(
        core_axis_name="c", subcore_axis_name="s",
        num_cores=_NC, num_subcores=_NS),
    compiler_params=pltpu.CompilerParams(
        use_tc_tiling_on_sc=False, needs_layout_passes=False),
    scratch_types=[
        pltpu.VMEM((_K,), jnp.int32),
        pltpu.VMEM((_K,), jnp.int32),
        pltpu.VMEM((_K, _ROW), jnp.float32),
        pltpu.VMEM((_K, _ERW), jnp.float32),
        pltpu.VMEM((_K, _ROW), jnp.float32),
        pltpu.VMEM((4, 16), jnp.float32),
        pltpu.VMEM_SHARED((_ACCN, _ROW), jnp.float32),
        pltpu.SemaphoreType.DMA,
        pltpu.SemaphoreType.DMA,
    ],
  )(_edge_body)


def _readout_body(acc_ref, gid_ref, w1_ref, b1_ref, w2_ref, b2_ref, out_ref):
    feats = []
    for c in range(2):
        a = acc_ref[c]
        for hh in range(4):
            den = a[:, _HALF + hh:_HALF + hh + 1] + 1e-9
            feats.append(a[:, hh * _F:(hh + 1) * _F] / den)
    feat = jnp.concatenate(feats, axis=1)
    gid = gid_ref[...]
    gi = lax.broadcasted_iota(jnp.int32, (_G, _N), 0)
    og = (jnp.broadcast_to(gid, (_G, _N)) == gi).astype(jnp.float32)
    sums = jnp.dot(og, feat, preferred_element_type=jnp.float32,
                   precision=lax.Precision.HIGHEST)
    cnts = jnp.sum(og, axis=1, keepdims=True)
    pooled = sums / jnp.maximum(cnts, 1.0)
    hid = jnp.maximum(
        jnp.dot(pooled, w1_ref[...], preferred_element_type=jnp.float32,
                precision=lax.Precision.HIGHEST)
        + b1_ref[...], 0.0)
    pred = jnp.sum(hid * w2_ref[...], axis=1, keepdims=True)
    out_ref[...] = pred + b2_ref[...]


_readout = pl.pallas_call(
    _readout_body,
    out_shape=jax.ShapeDtypeStruct((_G, 1), jnp.float32),
)


_DBG_JNP_READOUT = False
_DBG_JNP_EDGE = False


def kernel(x, edge_index, graph_ids, W, attn_l, attn_r, W1, b1, W2, b2):
    src = edge_index[0].astype(jnp.int32)
    dst = edge_index[1].astype(jnp.int32)
    blk = (jnp.arange(_HF)[:, None] // _F == jnp.arange(_H)[None, :])
    almat = jnp.where(blk, attn_l.reshape(_HF)[:, None], 0.0)
    armat = jnp.where(blk, attn_r.reshape(_HF)[:, None], 0.0)
    tlo, thi, ertab = _proj(x, W, almat, armat)
    if _DBG_JNP_EDGE:
        h2 = jnp.concatenate([tlo[:, :_HALF], thi[:, :_HALF]], 1)
        el2 = jnp.concatenate([tlo[:, _HALF:_HALF + 4],
                               thi[:, _HALF:_HALF + 4]], 1)
        er2 = ertab[:, :_H]
        e2 = el2[src] + er2[dst]
        e2 = jnp.where(e2 > 0, e2, _SLOPE * e2)
        ee = jnp.exp(e2)
        dn = jax.ops.segment_sum(ee, dst, num_segments=_N)
        nm = jax.ops.segment_sum(
            (ee[:, :, None] * h2[src].reshape(_E, _H, _F)).reshape(_E, _HF),
            dst, num_segments=_N)
        pad = jnp.zeros((_N, 12), jnp.float32)
        acc = jnp.stack([
            jnp.concatenate([nm[:, :_HALF], dn[:, :4], pad], 1),
            jnp.concatenate([nm[:, _HALF:], dn[:, 4:], pad], 1)])
    else:
        acc = _make_edge()(tlo, thi, ertab, src, dst)
    if _DBG_JNP_READOUT:
        num = jnp.concatenate([acc[0, :, :_HALF], acc[1, :, :_HALF]], 1)
        den = jnp.concatenate([acc[0, :, _HALF:_HALF + 4],
                               acc[1, :, _HALF:_HALF + 4]], 1)
        feat = num / (jnp.repeat(den, _F, axis=1) + 1e-9)
        sums = jax.ops.segment_sum(feat, graph_ids, num_segments=_G)
        cnts = jax.ops.segment_sum(jnp.ones((_N, 1), jnp.float32),
                                   graph_ids, num_segments=_G)
        pooled = sums / jnp.maximum(cnts, 1.0)
        hid = jax.nn.relu(pooled @ W1 + b1)
        return hid @ W2 + b2
    gid = graph_ids.astype(jnp.int32).reshape(1, _N)
    return _readout(acc, gid, W1, b1.reshape(1, _PH), W2.reshape(1, _PH),
                    b2.reshape(1, 1))
